# baseline (speedup 1.0000x reference)
import functools

import jax
import jax.numpy as jnp
from jax import lax
from jax.experimental import pallas as pl
from jax.experimental.pallas import tpu as pltpu
from jax.experimental.pallas import tpu_sc as plsc

N = 10000
E = 320000
D = 128
D_IN = 128
D_HID = 256
D_OUT = 128

NC = 2
NS = 16
NW = NC * NS
CH = 128
NG = 4
GC = 20
NCH = NG * GC
EPW = NCH * CH
EP = NW * EPW
RPT = 640
NACC = NS * RPT
NPADR = NACC - N

_MESH = plsc.VectorSubcoreMesh(
    core_axis_name="c", subcore_axis_name="s", num_cores=NC, num_subcores=NS
)


def _make_agg(with_counts):
    def body(*args):
        if with_counts:
            (tab_hbm, src_hbm, dst_hbm, zr_hbm, zc_hbm, out_hbm, cnt_hbm,
             sidx0_v, sidx1_v, didx0_v, didx1_v, rows0_v, rows1_v, ones_v,
             acc_sh, cnt_sh, gsem0, gsem1, ssem0, ssem1, isem, zsem) = args
        else:
            (tab_hbm, src_hbm, dst_hbm, zr_hbm, out_hbm,
             sidx0_v, sidx1_v, didx0_v, didx1_v, rows0_v, rows1_v,
             acc_sh, gsem0, gsem1, ssem0, ssem1, isem, zsem) = args
        c = lax.axis_index("c")
        s = lax.axis_index("s")
        wid = s * NC + c

        pltpu.async_copy(zr_hbm, acc_sh.at[pl.ds(s * RPT, RPT)], zsem)
        if with_counts:
            pltpu.async_copy(zc_hbm, cnt_sh.at[pl.ds(s * RPT, RPT)], zsem)
            for j in range(CH // 16):
                ones_v[pl.ds(j * 16, 16)] = jnp.ones((16,), jnp.float32)

        def gather(idx_v, i, buf, sem):
            pltpu.async_copy(tab_hbm.at[idx_v.at[i]], buf, sem)

        def wait(sem, buf):
            pltpu.make_async_copy(tab_hbm.at[pl.ds(0, CH)], buf, sem).wait()

        def scatter(idx_v, i, buf, sem):
            pltpu.async_copy(buf, acc_sh.at[idx_v.at[i]], sem, add=True)
            if with_counts:
                pltpu.sync_copy(ones_v, cnt_sh.at[idx_v.at[i]], add=True)

        def swait(sem, buf):
            pltpu.make_async_copy(tab_hbm.at[pl.ds(0, CH)], buf, sem).wait()

        sidx = (sidx0_v, sidx1_v)
        didx = (didx0_v, didx1_v)

        def load_idx(g):
            b = g % 2
            pltpu.async_copy(src_hbm.at[wid, g], sidx[b], isem)
            pltpu.async_copy(dst_hbm.at[wid, g], didx[b], isem)

        def wait_idx(g):
            b = g % 2
            pltpu.make_async_copy(src_hbm.at[wid, g], sidx[b], isem).wait()
            pltpu.make_async_copy(dst_hbm.at[wid, g], didx[b], isem).wait()

        load_idx(0)
        wait_idx(0)
        load_idx(1)
        gather(sidx[0], 0, rows0_v, gsem0)
        gather(sidx[0], 1, rows1_v, gsem1)
        pltpu.make_async_copy(zr_hbm, acc_sh.at[pl.ds(s * RPT, RPT)],
                              zsem).wait()
        if with_counts:
            pltpu.make_async_copy(zc_hbm, cnt_sh.at[pl.ds(s * RPT, RPT)],
                                  zsem).wait()
        plsc.subcore_barrier()

        for g in range(NG):
            sg, dg = sidx[g % 2], didx[g % 2]

            def pipelined(p, carry):
                i = 2 * p
                wait(gsem0, rows0_v)
                scatter(dg, i, rows0_v, ssem0)
                wait(gsem1, rows1_v)
                scatter(dg, i + 1, rows1_v, ssem1)
                swait(ssem0, rows0_v)
                gather(sg, i + 2, rows0_v, gsem0)
                swait(ssem1, rows1_v)
                gather(sg, i + 3, rows1_v, gsem1)
                return carry

            lax.fori_loop(0, GC // 2 - 1, pipelined, 0)
            wait(gsem0, rows0_v)
            scatter(dg, GC - 2, rows0_v, ssem0)
            wait(gsem1, rows1_v)
            scatter(dg, GC - 1, rows1_v, ssem1)
            if g + 1 < NG:
                wait_idx(g + 1)
                nsg = sidx[(g + 1) % 2]
                swait(ssem0, rows0_v)
                gather(nsg, 0, rows0_v, gsem0)
                swait(ssem1, rows1_v)
                gather(nsg, 1, rows1_v, gsem1)
                if g + 2 < NG:
                    load_idx(g + 2)
            else:
                swait(ssem0, rows0_v)
                swait(ssem1, rows1_v)

        plsc.subcore_barrier()
        pltpu.sync_copy(acc_sh.at[pl.ds(s * RPT, RPT)],
                        out_hbm.at[c, pl.ds(s * RPT, RPT)])
        if with_counts:
            pltpu.sync_copy(cnt_sh.at[pl.ds(s * RPT, RPT)],
                            cnt_hbm.at[c, pl.ds(s * RPT, RPT)])

    if with_counts:
        out_type = (jax.ShapeDtypeStruct((NC, NACC, D), jnp.float32),
                    jax.ShapeDtypeStruct((NC, NACC), jnp.float32))
        scratch = [
            pltpu.VMEM((GC, CH), jnp.int32),
            pltpu.VMEM((GC, CH), jnp.int32),
            pltpu.VMEM((GC, CH), jnp.int32),
            pltpu.VMEM((GC, CH), jnp.int32),
            pltpu.VMEM((CH, D), jnp.float32),
            pltpu.VMEM((CH, D), jnp.float32),
            pltpu.VMEM((CH,), jnp.float32),
            pltpu.VMEM_SHARED((NACC, D), jnp.float32),
            pltpu.VMEM_SHARED((NACC,), jnp.float32),
            pltpu.SemaphoreType.DMA,
            pltpu.SemaphoreType.DMA,
            pltpu.SemaphoreType.DMA,
            pltpu.SemaphoreType.DMA,
            pltpu.SemaphoreType.DMA,
            pltpu.SemaphoreType.DMA,
        ]
    else:
        out_type = jax.ShapeDtypeStruct((NC, NACC, D), jnp.float32)
        scratch = [
            pltpu.VMEM((GC, CH), jnp.int32),
            pltpu.VMEM((GC, CH), jnp.int32),
            pltpu.VMEM((GC, CH), jnp.int32),
            pltpu.VMEM((GC, CH), jnp.int32),
            pltpu.VMEM((CH, D), jnp.float32),
            pltpu.VMEM((CH, D), jnp.float32),
            pltpu.VMEM_SHARED((NACC, D), jnp.float32),
            pltpu.SemaphoreType.DMA,
            pltpu.SemaphoreType.DMA,
            pltpu.SemaphoreType.DMA,
            pltpu.SemaphoreType.DMA,
            pltpu.SemaphoreType.DMA,
            pltpu.SemaphoreType.DMA,
        ]
    return pl.kernel(body, out_type=out_type, mesh=_MESH,
                     scratch_types=scratch)


_agg_cnt = _make_agg(True)
_agg = _make_agg(False)


_MBLK = 1280


def _inv_cnt(cnt_block):
    cnts = cnt_block[0, :] + cnt_block[1, :]
    inv = 1.0 / jnp.maximum(cnts, 1.0)
    return inv[:, None]


def _mid_body(acc_ref, cnt_ref, x_ref, wl1_ref, bl1_ref, wr1_ref, wl2_ref,
              bl2_ref, wr2_ref, p2_ref, r2_ref):
    accs = acc_ref[0] + acc_ref[1]
    agg1 = accs * _inv_cnt(cnt_ref[...])
    h = (jnp.dot(agg1, wl1_ref[...], preferred_element_type=jnp.float32)
         + jnp.dot(x_ref[...], wr1_ref[...], preferred_element_type=jnp.float32)
         + bl1_ref[...])
    h = jnp.maximum(h, 0.0)
    p2_ref[...] = jnp.dot(h, wl2_ref[...], preferred_element_type=jnp.float32)
    r2_ref[...] = (jnp.dot(h, wr2_ref[...], preferred_element_type=jnp.float32)
                   + bl2_ref[...])


def _mid(acc1, cnt, x, Wl1, bl1, Wr1, Wl2, bl2, Wr2):
    grid = NACC // _MBLK
    return pl.pallas_call(
        _mid_body,
        grid=(grid,),
        in_specs=[
            pl.BlockSpec((NC, _MBLK, D), lambda i: (0, i, 0)),
            pl.BlockSpec((NC, _MBLK), lambda i: (0, i)),
            pl.BlockSpec((_MBLK, D_IN), lambda i: (i, 0)),
            pl.BlockSpec((D_IN, D_HID), lambda i: (0, 0)),
            pl.BlockSpec((1, D_HID), lambda i: (0, 0)),
            pl.BlockSpec((D_IN, D_HID), lambda i: (0, 0)),
            pl.BlockSpec((D_HID, D_OUT), lambda i: (0, 0)),
            pl.BlockSpec((1, D_OUT), lambda i: (0, 0)),
            pl.BlockSpec((D_HID, D_OUT), lambda i: (0, 0)),
        ],
        out_specs=[
            pl.BlockSpec((_MBLK, D_OUT), lambda i: (i, 0)),
            pl.BlockSpec((_MBLK, D_OUT), lambda i: (i, 0)),
        ],
        out_shape=[
            jax.ShapeDtypeStruct((N, D_OUT), jnp.float32),
            jax.ShapeDtypeStruct((N, D_OUT), jnp.float32),
        ],
    )(acc1, cnt, x, Wl1, bl1.reshape(1, D_HID), Wr1, Wl2,
      bl2.reshape(1, D_OUT), Wr2)


def _fin_body(acc_ref, cnt_ref, r2_ref, out_ref):
    accs = acc_ref[0] + acc_ref[1]
    out_ref[...] = accs * _inv_cnt(cnt_ref[...]) + r2_ref[...]


def _fin(acc2, cnt, r2):
    grid = NACC // _MBLK
    return pl.pallas_call(
        _fin_body,
        grid=(grid,),
        in_specs=[
            pl.BlockSpec((NC, _MBLK, D), lambda i: (0, i, 0)),
            pl.BlockSpec((NC, _MBLK), lambda i: (0, i)),
            pl.BlockSpec((_MBLK, D_OUT), lambda i: (i, 0)),
        ],
        out_specs=pl.BlockSpec((_MBLK, D_OUT), lambda i: (i, 0)),
        out_shape=jax.ShapeDtypeStruct((N, D_OUT), jnp.float32),
    )(acc2, cnt, r2)


def kernel(x, edge_index, Wl1, bl1, Wr1, Wl2, bl2, Wr2):
    npad = EP - E
    pad_src = (jnp.arange(npad, dtype=jnp.int32) * 131) % N
    pad_dst = N + (jnp.arange(npad, dtype=jnp.int32) % NPADR)
    src = jnp.concatenate([edge_index[0], pad_src]).reshape(NW, NG, GC, CH)
    dst = jnp.concatenate([edge_index[1], pad_dst]).reshape(NW, NG, GC, CH)
    zr = jnp.zeros((RPT, D), jnp.float32)
    zc = jnp.zeros((RPT,), jnp.float32)

    acc1, cnt = _agg_cnt(x, src, dst, zr, zc)
    p2, r2 = _mid(acc1, cnt, x, Wl1, bl1, Wr1, Wl2, bl2, Wr2)
    acc2 = _agg(p2, src, dst, zr)
    return _fin(acc2, cnt, r2)


# speedup vs baseline: 1.2749x; 1.2749x over previous
import functools

import jax
import jax.numpy as jnp
from jax import lax
from jax.experimental import pallas as pl
from jax.experimental.pallas import tpu as pltpu
from jax.experimental.pallas import tpu_sc as plsc

N = 10000
E = 320000
D = 128
D_IN = 128
D_HID = 256
D_OUT = 128

NC = 2
NS = 16
NW = NC * NS
CH = 128
NG = 4
GC = 20
NCH = NG * GC
EPW = NCH * CH
EP = NW * EPW
RPT = 640
NACC = NS * RPT
NPADR = NACC - N

_MESH = plsc.VectorSubcoreMesh(
    core_axis_name="c", subcore_axis_name="s", num_cores=NC, num_subcores=NS
)


def _make_agg(with_counts):
    def body(*args):
        if with_counts:
            (tab_hbm, src_hbm, dst_hbm, zr_hbm, zc_hbm, out_hbm, cnt_hbm,
             sidx0_v, sidx1_v, didx0_v, didx1_v, rows0_v, rows1_v, ones_v,
             acc_sh, cnt_sh, gsem0, gsem1, ssem0, ssem1, isem, zsem) = args
        else:
            (tab_hbm, src_hbm, dst_hbm, zr_hbm, out_hbm,
             sidx0_v, sidx1_v, didx0_v, didx1_v, rows0_v, rows1_v,
             acc_sh, gsem0, gsem1, ssem0, ssem1, isem, zsem) = args
        c = lax.axis_index("c")
        s = lax.axis_index("s")
        wid = s * NC + c

        pltpu.async_copy(zr_hbm, acc_sh.at[pl.ds(s * RPT, RPT)], zsem)
        if with_counts:
            pltpu.async_copy(zc_hbm, cnt_sh.at[pl.ds(s * RPT, RPT)], zsem)
            for j in range(CH // 16):
                ones_v[pl.ds(j * 16, 16)] = jnp.ones((16,), jnp.float32)

        def gather(idx_v, i, buf, sem):
            pltpu.async_copy(tab_hbm.at[idx_v.at[i]], buf, sem)

        def wait(sem, buf):
            pltpu.make_async_copy(tab_hbm.at[pl.ds(0, CH)], buf, sem).wait()

        def scatter(idx_v, i, buf):
            pltpu.sync_copy(buf, acc_sh.at[idx_v.at[i]], add=True)
            if with_counts:
                pltpu.sync_copy(ones_v, cnt_sh.at[idx_v.at[i]], add=True)

        sidx = (sidx0_v, sidx1_v)
        didx = (didx0_v, didx1_v)

        def load_idx(g):
            b = g % 2
            pltpu.async_copy(src_hbm.at[wid, g], sidx[b], isem)
            pltpu.async_copy(dst_hbm.at[wid, g], didx[b], isem)

        def wait_idx(g):
            b = g % 2
            pltpu.make_async_copy(src_hbm.at[wid, g], sidx[b], isem).wait()
            pltpu.make_async_copy(dst_hbm.at[wid, g], didx[b], isem).wait()

        load_idx(0)
        wait_idx(0)
        load_idx(1)
        gather(sidx[0], 0, rows0_v, gsem0)
        pltpu.make_async_copy(zr_hbm, acc_sh.at[pl.ds(s * RPT, RPT)],
                              zsem).wait()
        if with_counts:
            pltpu.make_async_copy(zc_hbm, cnt_sh.at[pl.ds(s * RPT, RPT)],
                                  zsem).wait()
        plsc.subcore_barrier()

        for g in range(NG):
            sg, dg = sidx[g % 2], didx[g % 2]

            def pipelined(p, carry):
                i = 2 * p
                gather(sg, i + 1, rows1_v, gsem1)
                wait(gsem0, rows0_v)
                scatter(dg, i, rows0_v)
                gather(sg, i + 2, rows0_v, gsem0)
                wait(gsem1, rows1_v)
                scatter(dg, i + 1, rows1_v)
                return carry

            lax.fori_loop(0, GC // 2 - 1, pipelined, 0)
            gather(sg, GC - 1, rows1_v, gsem1)
            wait(gsem0, rows0_v)
            scatter(dg, GC - 2, rows0_v)
            if g + 1 < NG:
                wait_idx(g + 1)
                gather(sidx[(g + 1) % 2], 0, rows0_v, gsem0)
            wait(gsem1, rows1_v)
            scatter(dg, GC - 1, rows1_v)
            if g + 2 < NG:
                load_idx(g + 2)

        plsc.subcore_barrier()
        pltpu.sync_copy(acc_sh.at[pl.ds(s * RPT, RPT)],
                        out_hbm.at[c, pl.ds(s * RPT, RPT)])
        if with_counts:
            pltpu.sync_copy(cnt_sh.at[pl.ds(s * RPT, RPT)],
                            cnt_hbm.at[c, pl.ds(s * RPT, RPT)])

    if with_counts:
        out_type = (jax.ShapeDtypeStruct((NC, NACC, D), jnp.float32),
                    jax.ShapeDtypeStruct((NC, NACC), jnp.float32))
        scratch = [
            pltpu.VMEM((GC, CH), jnp.int32),
            pltpu.VMEM((GC, CH), jnp.int32),
            pltpu.VMEM((GC, CH), jnp.int32),
            pltpu.VMEM((GC, CH), jnp.int32),
            pltpu.VMEM((CH, D), jnp.float32),
            pltpu.VMEM((CH, D), jnp.float32),
            pltpu.VMEM((CH,), jnp.float32),
            pltpu.VMEM_SHARED((NACC, D), jnp.float32),
            pltpu.VMEM_SHARED((NACC,), jnp.float32),
            pltpu.SemaphoreType.DMA,
            pltpu.SemaphoreType.DMA,
            pltpu.SemaphoreType.DMA,
            pltpu.SemaphoreType.DMA,
            pltpu.SemaphoreType.DMA,
            pltpu.SemaphoreType.DMA,
        ]
    else:
        out_type = jax.ShapeDtypeStruct((NC, NACC, D), jnp.float32)
        scratch = [
            pltpu.VMEM((GC, CH), jnp.int32),
            pltpu.VMEM((GC, CH), jnp.int32),
            pltpu.VMEM((GC, CH), jnp.int32),
            pltpu.VMEM((GC, CH), jnp.int32),
            pltpu.VMEM((CH, D), jnp.float32),
            pltpu.VMEM((CH, D), jnp.float32),
            pltpu.VMEM_SHARED((NACC, D), jnp.float32),
            pltpu.SemaphoreType.DMA,
            pltpu.SemaphoreType.DMA,
            pltpu.SemaphoreType.DMA,
            pltpu.SemaphoreType.DMA,
            pltpu.SemaphoreType.DMA,
            pltpu.SemaphoreType.DMA,
        ]
    return pl.kernel(body, out_type=out_type, mesh=_MESH,
                     scratch_types=scratch)


_agg_cnt = _make_agg(True)
_agg = _make_agg(False)


_MBLK = 1280


def _inv_cnt(cnt_block):
    cnts = cnt_block[0, :] + cnt_block[1, :]
    inv = 1.0 / jnp.maximum(cnts, 1.0)
    return inv[:, None]


def _mid_body(acc_ref, cnt_ref, x_ref, wl1_ref, bl1_ref, wr1_ref, wl2_ref,
              bl2_ref, wr2_ref, p2_ref, r2_ref):
    accs = acc_ref[0] + acc_ref[1]
    agg1 = accs * _inv_cnt(cnt_ref[...])
    h = (jnp.dot(agg1, wl1_ref[...], preferred_element_type=jnp.float32)
         + jnp.dot(x_ref[...], wr1_ref[...], preferred_element_type=jnp.float32)
         + bl1_ref[...])
    h = jnp.maximum(h, 0.0)
    p2_ref[...] = jnp.dot(h, wl2_ref[...], preferred_element_type=jnp.float32)
    r2_ref[...] = (jnp.dot(h, wr2_ref[...], preferred_element_type=jnp.float32)
                   + bl2_ref[...])


def _mid(acc1, cnt, x, Wl1, bl1, Wr1, Wl2, bl2, Wr2):
    grid = NACC // _MBLK
    return pl.pallas_call(
        _mid_body,
        grid=(grid,),
        in_specs=[
            pl.BlockSpec((NC, _MBLK, D), lambda i: (0, i, 0)),
            pl.BlockSpec((NC, _MBLK), lambda i: (0, i)),
            pl.BlockSpec((_MBLK, D_IN), lambda i: (i, 0)),
            pl.BlockSpec((D_IN, D_HID), lambda i: (0, 0)),
            pl.BlockSpec((1, D_HID), lambda i: (0, 0)),
            pl.BlockSpec((D_IN, D_HID), lambda i: (0, 0)),
            pl.BlockSpec((D_HID, D_OUT), lambda i: (0, 0)),
            pl.BlockSpec((1, D_OUT), lambda i: (0, 0)),
            pl.BlockSpec((D_HID, D_OUT), lambda i: (0, 0)),
        ],
        out_specs=[
            pl.BlockSpec((_MBLK, D_OUT), lambda i: (i, 0)),
            pl.BlockSpec((_MBLK, D_OUT), lambda i: (i, 0)),
        ],
        out_shape=[
            jax.ShapeDtypeStruct((N, D_OUT), jnp.float32),
            jax.ShapeDtypeStruct((N, D_OUT), jnp.float32),
        ],
    )(acc1, cnt, x, Wl1, bl1.reshape(1, D_HID), Wr1, Wl2,
      bl2.reshape(1, D_OUT), Wr2)


def _fin_body(acc_ref, cnt_ref, r2_ref, out_ref):
    accs = acc_ref[0] + acc_ref[1]
    out_ref[...] = accs * _inv_cnt(cnt_ref[...]) + r2_ref[...]


def _fin(acc2, cnt, r2):
    grid = NACC // _MBLK
    return pl.pallas_call(
        _fin_body,
        grid=(grid,),
        in_specs=[
            pl.BlockSpec((NC, _MBLK, D), lambda i: (0, i, 0)),
            pl.BlockSpec((NC, _MBLK), lambda i: (0, i)),
            pl.BlockSpec((_MBLK, D_OUT), lambda i: (i, 0)),
        ],
        out_specs=pl.BlockSpec((_MBLK, D_OUT), lambda i: (i, 0)),
        out_shape=jax.ShapeDtypeStruct((N, D_OUT), jnp.float32),
    )(acc2, cnt, r2)


def kernel(x, edge_index, Wl1, bl1, Wr1, Wl2, bl2, Wr2):
    npad = EP - E
    pad_src = (jnp.arange(npad, dtype=jnp.int32) * 131) % N
    pad_dst = N + (jnp.arange(npad, dtype=jnp.int32) % NPADR)
    src = jnp.concatenate([edge_index[0], pad_src]).reshape(NW, NG, GC, CH)
    dst = jnp.concatenate([edge_index[1], pad_dst]).reshape(NW, NG, GC, CH)
    zr = jnp.zeros((RPT, D), jnp.float32)
    zc = jnp.zeros((RPT,), jnp.float32)

    acc1, cnt = _agg_cnt(x, src, dst, zr, zc)
    p2, r2 = _mid(acc1, cnt, x, Wl1, bl1, Wr1, Wl2, bl2, Wr2)
    acc2 = _agg(p2, src, dst, zr)
    return _fin(acc2, cnt, r2)
